# baseline (speedup 1.0000x reference)
import functools

import jax
import jax.numpy as jnp
from jax import lax
from jax.experimental import pallas as pl
from jax.experimental.pallas import tpu as pltpu
from jax.experimental.pallas import tpu_sc as plsc

_D = 128
_SEQ = 512
_NROWS = 65
_NW = 32
_ROWS_PER_W = _SEQ // _NW
_G_PAD = 1024
_STRIPE = _G_PAD // 16
_TBL_SLICE = 33


def _sc_body(table_hbm, out_hbm, tbl_v, stripe_v, g_sh, gsem, osem):
    cid = lax.axis_index("c")
    sid = lax.axis_index("s")
    t0 = _STRIPE * sid
    pltpu.async_copy(table_hbm, tbl_v, gsem).wait()
    half = _STRIPE // 2
    for m in range(_STRIPE):
        if m == half:
            first = pltpu.async_copy(
                stripe_v.at[pl.ds(0, half)], g_sh.at[pl.ds(t0, half)], gsem
            )
        r = jnp.clip(511 - (t0 + m), -32, 32) + 32
        for q in range(_D // 16):
            stripe_v[m, pl.ds(q * 16, 16)] = tbl_v[r, pl.ds(q * 16, 16)]
    second = pltpu.async_copy(
        stripe_v.at[pl.ds(half, half)], g_sh.at[pl.ds(t0 + half, half)], gsem
    )
    first.wait()
    second.wait()
    plsc.subcore_barrier()

    wid = cid * 16 + sid
    base = wid * _ROWS_PER_W
    copies = []
    for k in range(_ROWS_PER_W):
        copies.append(
            pltpu.async_copy(
                g_sh.at[pl.ds(511 - (base + k), _SEQ)],
                out_hbm.at[pl.ds((base + k) * _SEQ, _SEQ)],
                osem,
            )
        )
    for c in copies:
        c.wait()


@jax.jit
def _rel_pos_sc(table):
    mesh = plsc.VectorSubcoreMesh(core_axis_name="c", subcore_axis_name="s")
    fn = functools.partial(
        pl.kernel,
        out_type=jax.ShapeDtypeStruct((_SEQ * _SEQ, _D), jnp.float32),
        mesh=mesh,
        scratch_types=[
            pltpu.VMEM((_NROWS, _D), jnp.float32),
            pltpu.VMEM((_STRIPE, _D), jnp.float32),
            pltpu.VMEM_SHARED((_G_PAD, _D), jnp.float32),
            pltpu.SemaphoreType.DMA,
            pltpu.SemaphoreType.DMA,
        ],
    )(_sc_body)
    return fn(table)


def kernel(seq_len, table):
    out = _rel_pos_sc(table)
    return out.reshape(_SEQ, _SEQ, _D)


# speedup vs baseline: 1.0005x; 1.0005x over previous
import functools

import jax
import jax.numpy as jnp
from jax import lax
from jax.experimental import pallas as pl
from jax.experimental.pallas import tpu as pltpu
from jax.experimental.pallas import tpu_sc as plsc

_D = 128
_SEQ = 512
_NROWS = 65
_NW = 32
_ROWS_PER_W = _SEQ // _NW
_G_PAD = 1024
_STRIPE = _G_PAD // 16


def _sc_body(table_hbm, out_hbm, tbl_v, stripe_v, g_sh, gsem, osem):
    cid = lax.axis_index("c")
    sid = lax.axis_index("s")
    t0 = _STRIPE * sid
    pltpu.async_copy(table_hbm, tbl_v, gsem).wait()
    half = _STRIPE // 2
    for m in range(_STRIPE):
        if m == half:
            first = pltpu.async_copy(
                stripe_v.at[pl.ds(0, half)], g_sh.at[pl.ds(t0, half)], gsem
            )
        r = jnp.clip(511 - (t0 + m), -32, 32) + 32
        for q in range(_D // 16):
            stripe_v[m, pl.ds(q * 16, 16)] = tbl_v[r, pl.ds(q * 16, 16)]
    second = pltpu.async_copy(
        stripe_v.at[pl.ds(half, half)], g_sh.at[pl.ds(t0 + half, half)], gsem
    )
    first.wait()
    second.wait()
    plsc.subcore_barrier()

    wid = cid * 16 + sid
    base = wid * _ROWS_PER_W
    copies = []
    for k in range(_ROWS_PER_W):
        copies.append(
            pltpu.async_copy(
                g_sh.at[pl.ds(511 - (base + k), _SEQ)],
                out_hbm.at[pl.ds((base + k) * _SEQ, _SEQ)],
                osem,
            )
        )
    for c in copies:
        c.wait()


@jax.jit
def _rel_pos_sc(table):
    mesh = plsc.VectorSubcoreMesh(core_axis_name="c", subcore_axis_name="s")
    fn = functools.partial(
        pl.kernel,
        out_type=jax.ShapeDtypeStruct((_SEQ * _SEQ, _D), jnp.float32),
        mesh=mesh,
        scratch_types=[
            pltpu.VMEM((_NROWS, _D), jnp.float32),
            pltpu.VMEM((_STRIPE, _D), jnp.float32),
            pltpu.VMEM_SHARED((_G_PAD, _D), jnp.float32),
            pltpu.SemaphoreType.DMA,
            pltpu.SemaphoreType.DMA,
        ],
    )(_sc_body)
    return fn(table)


def kernel(seq_len, table):
    out = _rel_pos_sc(table)
    return out.reshape(_SEQ, _SEQ, _D)


# speedup vs baseline: 1.0260x; 1.0255x over previous
import functools

import jax
import jax.numpy as jnp
from jax import lax
from jax.experimental import pallas as pl
from jax.experimental.pallas import tpu as pltpu
from jax.experimental.pallas import tpu_sc as plsc

_D = 128
_SEQ = 512
_NROWS = 65
_NW = 32
_ROWS_PER_W = _SEQ // _NW
_G_PAD = 1024
_STRIPE = _G_PAD // 16


def _sc_body(table_hbm, out_hbm, tbl_v, stripe_v, g_sh, gsem, osem):
    cid = lax.axis_index("c")
    sid = lax.axis_index("s")
    t0 = _STRIPE * sid
    pltpu.async_copy(table_hbm, tbl_v, gsem).wait()
    half = _STRIPE // 2

    def _build_row(m, _):
        r = jnp.clip(511 - (t0 + m), -32, 32) + 32
        for q in range(_D // 16):
            stripe_v[m, pl.ds(q * 16, 16)] = tbl_v[r, pl.ds(q * 16, 16)]
        return _

    lax.fori_loop(0, half, _build_row, None)
    first = pltpu.async_copy(
        stripe_v.at[pl.ds(0, half)], g_sh.at[pl.ds(t0, half)], gsem
    )
    lax.fori_loop(half, _STRIPE, _build_row, None)
    second = pltpu.async_copy(
        stripe_v.at[pl.ds(half, half)], g_sh.at[pl.ds(t0 + half, half)], gsem
    )
    first.wait()
    second.wait()
    plsc.subcore_barrier()

    wid = cid * 16 + sid
    base = wid * _ROWS_PER_W
    copies = []
    for k in range(_ROWS_PER_W):
        copies.append(
            pltpu.async_copy(
                g_sh.at[pl.ds(511 - (base + k), _SEQ)],
                out_hbm.at[pl.ds((base + k) * _SEQ, _SEQ)],
                osem,
            )
        )
    for c in copies:
        c.wait()


@jax.jit
def _rel_pos_sc(table):
    mesh = plsc.VectorSubcoreMesh(core_axis_name="c", subcore_axis_name="s")
    fn = functools.partial(
        pl.kernel,
        out_type=jax.ShapeDtypeStruct((_SEQ * _SEQ, _D), jnp.float32),
        mesh=mesh,
        scratch_types=[
            pltpu.VMEM((_NROWS, _D), jnp.float32),
            pltpu.VMEM((_STRIPE, _D), jnp.float32),
            pltpu.VMEM_SHARED((_G_PAD, _D), jnp.float32),
            pltpu.SemaphoreType.DMA,
            pltpu.SemaphoreType.DMA,
        ],
    )(_sc_body)
    return fn(table)


def kernel(seq_len, table):
    out = _rel_pos_sc(table)
    return out.reshape(_SEQ, _SEQ, _D)


# speedup vs baseline: 1.0267x; 1.0006x over previous
import functools

import jax
import jax.numpy as jnp
from jax import lax
from jax.experimental import pallas as pl
from jax.experimental.pallas import tpu as pltpu
from jax.experimental.pallas import tpu_sc as plsc

_D = 128
_SEQ = 512
_NROWS = 65
_NW = 32
_ROWS_PER_W = _SEQ // _NW
_G_PAD = 1024
_STRIPE = _G_PAD // 16


def _sc_body(table_hbm, out_hbm, tbl_v, stripe_v, g_sh, gsem, osem):
    cid = lax.axis_index("c")
    sid = lax.axis_index("s")
    t0 = _STRIPE * sid
    pltpu.async_copy(table_hbm, tbl_v, gsem).wait()
    half = _STRIPE // 2

    def _build_row(m, _):
        r = jnp.clip(511 - (t0 + m), -32, 32) + 32
        for q in range(_D // 16):
            stripe_v[m, pl.ds(q * 16, 16)] = tbl_v[r, pl.ds(q * 16, 16)]
        return _

    lax.fori_loop(0, half, _build_row, None)
    first = pltpu.async_copy(
        stripe_v.at[pl.ds(0, half)], g_sh.at[pl.ds(t0, half)], gsem
    )
    lax.fori_loop(half, _STRIPE, _build_row, None)
    second = pltpu.async_copy(
        stripe_v.at[pl.ds(half, half)], g_sh.at[pl.ds(t0 + half, half)], gsem
    )
    first.wait()
    second.wait()
    plsc.subcore_barrier()

    wid = cid * 16 + sid
    base = wid * _ROWS_PER_W

    def _issue_row(k, _):
        i = base + k
        pltpu.async_copy(
            g_sh.at[pl.ds(511 - i, _SEQ)],
            out_hbm.at[pl.ds(i * _SEQ, _SEQ)],
            osem,
        )
        return _

    lax.fori_loop(0, _ROWS_PER_W, _issue_row, None)

    def _drain_row(k, _):
        pltpu.make_async_copy(
            g_sh.at[pl.ds(0, _SEQ)],
            out_hbm.at[pl.ds(base * _SEQ, _SEQ)],
            osem,
        ).wait()
        return _

    lax.fori_loop(0, _ROWS_PER_W, _drain_row, None)


@jax.jit
def _rel_pos_sc(table):
    mesh = plsc.VectorSubcoreMesh(core_axis_name="c", subcore_axis_name="s")
    fn = functools.partial(
        pl.kernel,
        out_type=jax.ShapeDtypeStruct((_SEQ * _SEQ, _D), jnp.float32),
        mesh=mesh,
        scratch_types=[
            pltpu.VMEM((_NROWS, _D), jnp.float32),
            pltpu.VMEM((_STRIPE, _D), jnp.float32),
            pltpu.VMEM_SHARED((_G_PAD, _D), jnp.float32),
            pltpu.SemaphoreType.DMA,
            pltpu.SemaphoreType.DMA,
        ],
    )(_sc_body)
    return fn(table)


def kernel(seq_len, table):
    out = _rel_pos_sc(table)
    return out.reshape(_SEQ, _SEQ, _D)


# speedup vs baseline: 1.0295x; 1.0028x over previous
import functools

import jax
import jax.numpy as jnp
from jax import lax
from jax.experimental import pallas as pl
from jax.experimental.pallas import tpu as pltpu
from jax.experimental.pallas import tpu_sc as plsc

_D = 128
_SEQ = 512
_NROWS = 65
_NW = 32
_ROWS_PER_W = _SEQ // _NW
_G_PAD = 1024
_STRIPE = _G_PAD // 16


def _sc_body(table_hbm, out_hbm, tbl_v, stripe_v, g_sh, gsem, osem):
    cid = lax.axis_index("c")
    sid = lax.axis_index("s")
    t0 = _STRIPE * sid
    pltpu.async_copy(table_hbm, tbl_v, gsem).wait()
    half = _STRIPE // 2

    def _build_row(m, _):
        r = jnp.clip(511 - (t0 + m), -32, 32) + 32
        for q in range(_D // 16):
            stripe_v[m, pl.ds(q * 16, 16)] = tbl_v[r, pl.ds(q * 16, 16)]
        return _

    lax.fori_loop(0, half, _build_row, None)
    first = pltpu.async_copy(
        stripe_v.at[pl.ds(0, half)], g_sh.at[pl.ds(t0, half)], gsem
    )
    lax.fori_loop(half, _STRIPE, _build_row, None)
    second = pltpu.async_copy(
        stripe_v.at[pl.ds(half, half)], g_sh.at[pl.ds(t0 + half, half)], gsem
    )
    first.wait()
    second.wait()
    plsc.subcore_barrier()

    wid = cid * 16 + sid
    base = wid * _ROWS_PER_W
    copies = []
    for k in range(_ROWS_PER_W):
        copies.append(
            pltpu.async_copy(
                g_sh.at[pl.ds(511 - (base + k), _SEQ)],
                out_hbm.at[pl.ds((base + k) * _SEQ, _SEQ)],
                osem,
            )
        )
    for c in copies:
        c.wait()


@jax.jit
def _rel_pos_sc(table):
    mesh = plsc.VectorSubcoreMesh(core_axis_name="c", subcore_axis_name="s")
    fn = functools.partial(
        pl.kernel,
        out_type=jax.ShapeDtypeStruct((_SEQ * _SEQ, _D), jnp.float32),
        mesh=mesh,
        scratch_types=[
            pltpu.VMEM((_NROWS, _D), jnp.float32),
            pltpu.VMEM((_STRIPE, _D), jnp.float32),
            pltpu.VMEM_SHARED((_G_PAD, _D), jnp.float32),
            pltpu.SemaphoreType.DMA,
            pltpu.SemaphoreType.DMA,
        ],
    )(_sc_body)
    return fn(table)


def kernel(seq_len, table):
    out = _rel_pos_sc(table)
    return out.reshape(_SEQ, _SEQ, _D)
